# Initial kernel scaffold; baseline (speedup 1.0000x reference)
import jax
import jax.numpy as jnp
from jax.experimental import pallas as pl


def kernel(x, edge_index, edge_weight, W, b):
    raise NotImplementedError("write your pallas kernel here")


# speedup vs baseline: 5.1021x; 5.1021x over previous
import functools

import jax
import jax.numpy as jnp
from jax import lax
from jax.experimental import pallas as pl
from jax.experimental.pallas import tpu as pltpu
from jax.experimental.pallas import tpu_sc as plsc

N_NODES = 10000
N_EDGES = 320000
DIM = 128

NC = 2
NS = 16
NW = NC * NS
CB = 128
T = -(-N_EDGES // (NW * CB))
E_PAD = NW * T * CB
N_ACC = 10240
ROWS_PER_TILE = N_ACC // NS


def _sc_spmm(src3, dst3, w3, x):
    mesh = plsc.VectorSubcoreMesh(core_axis_name="c", subcore_axis_name="s")

    @functools.partial(
        pl.kernel,
        mesh=mesh,
        out_type=jax.ShapeDtypeStruct((NC, N_ACC, DIM), jnp.float32),
        scratch_types=[
            pltpu.VMEM((T, CB), jnp.int32),
            pltpu.VMEM((T, CB), jnp.int32),
            pltpu.VMEM((T, CB), jnp.float32),
            pltpu.VMEM((CB, DIM), jnp.float32),
            pltpu.VMEM_SHARED((N_ACC, DIM), jnp.float32),
            pltpu.SemaphoreType.DMA,
        ],
    )
    def body(src_hbm, dst_hbm, w_hbm, x_hbm, out_hbm,
             src_v, dst_v, w_v, rows_v, acc_sh, sem):
        cid = lax.axis_index("c")
        sid = lax.axis_index("s")
        wid = cid * NS + sid

        pltpu.sync_copy(src_hbm.at[wid], src_v)
        pltpu.sync_copy(dst_hbm.at[wid], dst_v)
        pltpu.sync_copy(w_hbm.at[wid], w_v)

        def zero_row(r, _):
            for c in range(DIM // 16):
                rows_v[r, pl.ds(c * 16, 16)] = jnp.zeros((16,), jnp.float32)
            return _
        lax.fori_loop(0, CB, zero_row, None)
        for k in range(ROWS_PER_TILE // CB):
            pltpu.sync_copy(
                rows_v,
                acc_sh.at[pl.ds(sid * ROWS_PER_TILE + k * CB, CB)])
        plsc.subcore_barrier()

        def chunk(j, _):
            pltpu.async_copy(x_hbm.at[src_v.at[j]], rows_v, sem).wait()
            def scale_group(g, _):
                wv = w_v[j, pl.ds(g * 16, 16)]
                for l in range(16):
                    wr = wv[l]
                    r = g * 16 + l
                    for c in range(DIM // 16):
                        sl = pl.ds(c * 16, 16)
                        rows_v[r, sl] = rows_v[r, sl] * wr
                return _
            lax.fori_loop(0, CB // 16, scale_group, None)
            pltpu.sync_copy(rows_v, acc_sh.at[dst_v.at[j]], add=True)
            return _
        lax.fori_loop(0, T, chunk, None)

        plsc.subcore_barrier()
        pltpu.sync_copy(
            acc_sh.at[pl.ds(sid * ROWS_PER_TILE, ROWS_PER_TILE)],
            out_hbm.at[cid, pl.ds(sid * ROWS_PER_TILE, ROWS_PER_TILE)])

    return body(src3, dst3, w3, x)


_BM = 512


def _mm_body(p_ref, w_ref, b_ref, o_ref):
    s = p_ref[0] + p_ref[1]
    o_ref[...] = (
        jnp.dot(s, w_ref[...], preferred_element_type=jnp.float32)
        + b_ref[...])


def _tc_combine_mm(partials, W, b):
    return pl.pallas_call(
        _mm_body,
        grid=(N_ACC // _BM,),
        in_specs=[
            pl.BlockSpec((NC, _BM, DIM), lambda i: (0, i, 0)),
            pl.BlockSpec((DIM, DIM), lambda i: (0, 0)),
            pl.BlockSpec((1, DIM), lambda i: (0, 0)),
        ],
        out_specs=pl.BlockSpec((_BM, DIM), lambda i: (i, 0)),
        out_shape=jax.ShapeDtypeStruct((N_ACC, DIM), jnp.float32),
    )(partials, W, b.reshape(1, DIM))


def kernel(x, edge_index, edge_weight, W, b):
    src = edge_index[0].astype(jnp.int32)
    dst = edge_index[1].astype(jnp.int32)
    w = edge_weight.astype(jnp.float32)
    pad = E_PAD - N_EDGES
    src3 = jnp.pad(src, (0, pad)).reshape(NW, T, CB)
    dst3 = jnp.pad(dst, (0, pad)).reshape(NW, T, CB)
    w3 = jnp.pad(w, (0, pad)).reshape(NW, T, CB)
    partials = _sc_spmm(src3, dst3, w3, x)
    return _tc_combine_mm(partials, W, b)[:N_NODES]
